# baseline (speedup 1.0000x reference)
import functools

import jax
import jax.numpy as jnp
from jax import lax
from jax.experimental import pallas as pl
from jax.experimental.pallas import tpu as pltpu
from jax.experimental.pallas import tpu_sc as plsc

NUM_CORES = 2
NUM_SUBCORES = 16
LANES = 16
NUM_WORKERS = NUM_CORES * NUM_SUBCORES

BATCH = 16384
E_DIM = 64
BPW = BATCH // NUM_WORKERS
GROUPS = BPW // LANES


def _score_kernel(trip_hbm, ent_hbm, rel_hbm, out_hbm,
                  trip_v, sidx_v, pidx_v, oidx_v, s_rows, p_rows, o_rows,
                  out_v, sem):
    wid = lax.axis_index("s") * NUM_CORES + lax.axis_index("c")
    base = wid * BPW

    pltpu.sync_copy(trip_hbm.at[pl.ds(base * 3, BPW * 3)], trip_v)
    lane_iota = lax.iota(jnp.int32, LANES)

    def deint_body(g, carry):
        rvec3 = (g * LANES + lane_iota) * 3
        sl = pl.ds(g * LANES, LANES)
        sidx_v[sl] = plsc.load_gather(trip_v, [rvec3])
        pidx_v[sl] = plsc.load_gather(trip_v, [rvec3 + 1])
        oidx_v[sl] = plsc.load_gather(trip_v, [rvec3 + 2])
        return carry

    lax.fori_loop(0, GROUPS, deint_body, 0)

    cp_s = pltpu.make_async_copy(ent_hbm.at[sidx_v], s_rows, sem)
    cp_p = pltpu.make_async_copy(rel_hbm.at[pidx_v], p_rows, sem)
    cp_o = pltpu.make_async_copy(ent_hbm.at[oidx_v], o_rows, sem)
    cp_s.start()
    cp_p.start()
    cp_o.start()
    cp_s.wait()
    cp_p.wait()
    cp_o.wait()

    def group_body(g, carry):
        rvec = g * LANES + lane_iota
        acc = jnp.zeros((LANES,), jnp.float32)
        for d in range(E_DIM):
            dvec = jnp.full((LANES,), d, jnp.int32)
            sv = plsc.load_gather(s_rows, [rvec, dvec])
            pv = plsc.load_gather(p_rows, [rvec, dvec])
            ov = plsc.load_gather(o_rows, [rvec, dvec])
            acc = acc + sv * pv * ov
        out_v[pl.ds(g * LANES, LANES)] = 1.0 / (1.0 + jnp.exp(-acc))
        return carry

    lax.fori_loop(0, GROUPS, group_body, 0)
    pltpu.sync_copy(out_v, out_hbm.at[pl.ds(base, BPW)])


@jax.jit
def _score(trip_flat, ent_table, rel_table):
    mesh = plsc.VectorSubcoreMesh(core_axis_name="c", subcore_axis_name="s")
    run = functools.partial(
        pl.kernel,
        out_type=jax.ShapeDtypeStruct((BATCH,), jnp.float32),
        mesh=mesh,
        compiler_params=pltpu.CompilerParams(
            needs_layout_passes=False, use_tc_tiling_on_sc=False),
        scratch_types=[
            pltpu.VMEM((BPW * 3,), jnp.int32),
            pltpu.VMEM((BPW,), jnp.int32),
            pltpu.VMEM((BPW,), jnp.int32),
            pltpu.VMEM((BPW,), jnp.int32),
            pltpu.VMEM((BPW, E_DIM), jnp.float32),
            pltpu.VMEM((BPW, E_DIM), jnp.float32),
            pltpu.VMEM((BPW, E_DIM), jnp.float32),
            pltpu.VMEM((BPW,), jnp.float32),
            pltpu.SemaphoreType.DMA,
        ],
    )(_score_kernel)
    return run(trip_flat, ent_table, rel_table)


def kernel(inputs, ent_table, rel_table):
    trip_flat = inputs.astype(jnp.int32).reshape(-1)
    score = _score(trip_flat, ent_table, rel_table)
    return score[:, None]


# speedup vs baseline: 1.0048x; 1.0048x over previous
import functools

import jax
import jax.numpy as jnp
from jax import lax
from jax.experimental import pallas as pl
from jax.experimental.pallas import tpu as pltpu
from jax.experimental.pallas import tpu_sc as plsc

NUM_CORES = 2
NUM_SUBCORES = 16
LANES = 16
NUM_WORKERS = NUM_CORES * NUM_SUBCORES

BATCH = 16384
E_DIM = 64
BPW = BATCH // NUM_WORKERS
GROUPS = BPW // LANES


def _score_kernel(sidx_hbm, pidx_hbm, oidx_hbm, ent_hbm, rel_hbm, out_hbm,
                  sidx_v, pidx_v, oidx_v, s_rows, p_rows, o_rows,
                  out_v, sem):
    wid = lax.axis_index("s") * NUM_CORES + lax.axis_index("c")
    base = wid * BPW

    pltpu.sync_copy(sidx_hbm.at[pl.ds(base, BPW)], sidx_v)
    pltpu.sync_copy(pidx_hbm.at[pl.ds(base, BPW)], pidx_v)
    pltpu.sync_copy(oidx_hbm.at[pl.ds(base, BPW)], oidx_v)
    lane_iota = lax.iota(jnp.int32, LANES)

    cp_s = pltpu.make_async_copy(ent_hbm.at[sidx_v], s_rows, sem)
    cp_p = pltpu.make_async_copy(rel_hbm.at[pidx_v], p_rows, sem)
    cp_o = pltpu.make_async_copy(ent_hbm.at[oidx_v], o_rows, sem)
    cp_s.start()
    cp_p.start()
    cp_o.start()
    cp_s.wait()
    cp_p.wait()
    cp_o.wait()

    def group_body(g, carry):
        rvec = g * LANES + lane_iota
        acc = jnp.zeros((LANES,), jnp.float32)
        for d in range(E_DIM):
            dvec = jnp.full((LANES,), d, jnp.int32)
            sv = plsc.load_gather(s_rows, [rvec, dvec])
            pv = plsc.load_gather(p_rows, [rvec, dvec])
            ov = plsc.load_gather(o_rows, [rvec, dvec])
            acc = acc + sv * pv * ov
        out_v[pl.ds(g * LANES, LANES)] = 1.0 / (1.0 + jnp.exp(-acc))
        return carry

    lax.fori_loop(0, GROUPS, group_body, 0)
    pltpu.sync_copy(out_v, out_hbm.at[pl.ds(base, BPW)])


@jax.jit
def _score(s_idx, p_idx, o_idx, ent_table, rel_table):
    mesh = plsc.VectorSubcoreMesh(core_axis_name="c", subcore_axis_name="s")
    run = functools.partial(
        pl.kernel,
        out_type=jax.ShapeDtypeStruct((BATCH,), jnp.float32),
        mesh=mesh,
        compiler_params=pltpu.CompilerParams(
            needs_layout_passes=False, use_tc_tiling_on_sc=False),
        scratch_types=[
            pltpu.VMEM((BPW,), jnp.int32),
            pltpu.VMEM((BPW,), jnp.int32),
            pltpu.VMEM((BPW,), jnp.int32),
            pltpu.VMEM((BPW, E_DIM), jnp.float32),
            pltpu.VMEM((BPW, E_DIM), jnp.float32),
            pltpu.VMEM((BPW, E_DIM), jnp.float32),
            pltpu.VMEM((BPW,), jnp.float32),
            pltpu.SemaphoreType.DMA,
        ],
    )(_score_kernel)
    return run(s_idx, p_idx, o_idx, ent_table, rel_table)


def kernel(inputs, ent_table, rel_table):
    idx = inputs.astype(jnp.int32)
    s_idx = jnp.bitwise_and(idx[:, 0], 0x7FFFFFFF)
    p_idx = jnp.bitwise_and(idx[:, 1], 0x7FFFFFFF)
    o_idx = jnp.bitwise_and(idx[:, 2], 0x7FFFFFFF)
    score = _score(s_idx, p_idx, o_idx, ent_table, rel_table)
    return score[:, None]
